# Initial kernel scaffold; baseline (speedup 1.0000x reference)
import functools

import jax
import jax.numpy as jnp
from jax import lax
from jax.experimental import pallas as pl
from jax.experimental.pallas import tpu as pltpu
from jax.experimental.pallas import tpu_sc as plsc

NUM_EMB = 1_000_000
DIM = 32
ROWS = 16384
COLS = 26
B_TOTAL = ROWS * COLS

NC = 2
NS = 16
NW = NC * NS
B_PER_W = B_TOTAL // NW
CHUNK = 1024
NCHUNK = B_PER_W // CHUNK

_mesh = plsc.VectorSubcoreMesh(core_axis_name="c", subcore_axis_name="s")


@functools.partial(
    pl.kernel,
    mesh=_mesh,
    out_type=jax.ShapeDtypeStruct((B_TOTAL, DIM), jnp.float32),
    scratch_types=[
        pltpu.VMEM((CHUNK,), jnp.int32),
        pltpu.VMEM((CHUNK, DIM), jnp.float32),
        pltpu.SemaphoreType.DMA,
    ],
)
def _gather_kernel(idx_hbm, table_hbm, out_hbm, idx_v, rows_v, sem):
    wid = lax.axis_index("s") * NC + lax.axis_index("c")
    wbase = wid * B_PER_W

    def body(i, carry):
        base = wbase + i * CHUNK
        pltpu.sync_copy(idx_hbm.at[pl.ds(base, CHUNK)], idx_v)
        pltpu.async_copy(table_hbm.at[idx_v], rows_v, sem).wait()
        pltpu.sync_copy(rows_v, out_hbm.at[pl.ds(base, CHUNK)])
        return carry

    lax.fori_loop(0, NCHUNK, body, 0)


def kernel(X, weight):
    idx = X.reshape(-1).astype(jnp.int32)
    out = _gather_kernel(idx, weight)
    return out.reshape(ROWS, COLS, DIM)


# speedup vs baseline: 1.5467x; 1.5467x over previous
import functools

import jax
import jax.numpy as jnp
from jax import lax
from jax.experimental import pallas as pl
from jax.experimental.pallas import tpu as pltpu
from jax.experimental.pallas import tpu_sc as plsc

NUM_EMB = 1_000_000
DIM = 32
ROWS = 16384
COLS = 26
B_TOTAL = ROWS * COLS

NC = 2
NS = 16
NW = NC * NS
B_PER_W = B_TOTAL // NW
CHUNK = 1024
NCHUNK = B_PER_W // CHUNK

_mesh = plsc.VectorSubcoreMesh(core_axis_name="c", subcore_axis_name="s")


@functools.partial(
    pl.kernel,
    mesh=_mesh,
    out_type=jax.ShapeDtypeStruct((B_TOTAL, DIM), jnp.float32),
    scratch_types=[
        pltpu.VMEM((CHUNK,), jnp.int32),
        pltpu.VMEM((CHUNK, DIM), jnp.float32),
        pltpu.SemaphoreType.DMA,
    ],
    compiler_params=pltpu.CompilerParams(use_tc_tiling_on_sc=False),
)
def _gather_kernel(idx_hbm, table_hbm, out_hbm, idx_v, rows_v, sem):
    wid = lax.axis_index("s") * NC + lax.axis_index("c")
    wbase = wid * B_PER_W

    def body(i, carry):
        base = wbase + i * CHUNK
        pltpu.sync_copy(idx_hbm.at[pl.ds(base, CHUNK)], idx_v)
        pltpu.async_copy(table_hbm.at[idx_v], rows_v, sem).wait()
        pltpu.sync_copy(rows_v, out_hbm.at[pl.ds(base, CHUNK)])
        return carry

    lax.fori_loop(0, NCHUNK, body, 0)


def kernel(X, weight):
    idx = X.reshape(-1).astype(jnp.int32)
    out = _gather_kernel(idx, weight)
    return out.reshape(ROWS, COLS, DIM)


# speedup vs baseline: 1.5745x; 1.0180x over previous
import functools

import jax
import jax.numpy as jnp
from jax import lax
from jax.experimental import pallas as pl
from jax.experimental.pallas import tpu as pltpu
from jax.experimental.pallas import tpu_sc as plsc

NUM_EMB = 1_000_000
DIM = 32
ROWS = 16384
COLS = 26
B_TOTAL = ROWS * COLS

NC = 2
NS = 16
NW = NC * NS
B_PER_W = B_TOTAL // NW
CHUNK = 1664
NCHUNK = B_PER_W // CHUNK

_mesh = plsc.VectorSubcoreMesh(core_axis_name="c", subcore_axis_name="s")


@functools.partial(
    pl.kernel,
    mesh=_mesh,
    out_type=jax.ShapeDtypeStruct((B_TOTAL, DIM), jnp.float32),
    scratch_types=[
        pltpu.VMEM((B_PER_W,), jnp.int32),
        pltpu.VMEM((CHUNK, DIM), jnp.float32),
        pltpu.VMEM((CHUNK, DIM), jnp.float32),
        pltpu.SemaphoreType.DMA,
        pltpu.SemaphoreType.DMA,
        pltpu.SemaphoreType.DMA,
        pltpu.SemaphoreType.DMA,
    ],
    compiler_params=pltpu.CompilerParams(use_tc_tiling_on_sc=False),
)
def _gather_kernel(idx_hbm, table_hbm, out_hbm, idx_all, rows0, rows1,
                   sg0, sg1, ss0, ss1):
    wid = lax.axis_index("s") * NC + lax.axis_index("c")
    wbase = wid * B_PER_W
    rows = [rows0, rows1]
    sg = [sg0, sg1]
    ss = [ss0, ss1]

    pltpu.sync_copy(idx_hbm.at[pl.ds(wbase, B_PER_W)], idx_all)

    def idx_of(c):
        return idx_all.at[pl.ds(c * CHUNK, CHUNK)]

    def out_of(c):
        return out_hbm.at[pl.ds(wbase + c * CHUNK, CHUNK)]

    pltpu.async_copy(table_hbm.at[idx_of(0)], rows[0], sg[0])

    for c in range(NCHUNK):
        b = c % 2
        o = b ^ 1
        if c + 1 < NCHUNK:
            if c - 1 >= 0:
                pltpu.make_async_copy(rows[o], out_of(c - 1), ss[o]).wait()
            pltpu.async_copy(table_hbm.at[idx_of(c + 1)], rows[o], sg[o])
        pltpu.make_async_copy(table_hbm.at[idx_of(c)], rows[b], sg[b]).wait()
        pltpu.async_copy(rows[b], out_of(c), ss[b])

    pltpu.make_async_copy(rows[(NCHUNK - 2) % 2], out_of(NCHUNK - 2),
                          ss[(NCHUNK - 2) % 2]).wait()
    pltpu.make_async_copy(rows[(NCHUNK - 1) % 2], out_of(NCHUNK - 1),
                          ss[(NCHUNK - 1) % 2]).wait()


def kernel(X, weight):
    idx = X.reshape(-1).astype(jnp.int32)
    out = _gather_kernel(idx, weight)
    return out.reshape(ROWS, COLS, DIM)


# speedup vs baseline: 1.6737x; 1.0630x over previous
import functools

import jax
import jax.numpy as jnp
from jax import lax
from jax.experimental import pallas as pl
from jax.experimental.pallas import tpu as pltpu
from jax.experimental.pallas import tpu_sc as plsc

NUM_EMB = 1_000_000
DIM = 32
ROWS = 16384
COLS = 26
B_TOTAL = ROWS * COLS

NC = 2
NS = 16
NW = NC * NS
B_PER_W = B_TOTAL // NW
CHUNK = 1664
NCHUNK = B_PER_W // CHUNK

_mesh = plsc.VectorSubcoreMesh(core_axis_name="c", subcore_axis_name="s")


@functools.partial(
    pl.kernel,
    mesh=_mesh,
    out_type=jax.ShapeDtypeStruct((B_TOTAL, DIM), jnp.float32),
    scratch_types=[
        pltpu.VMEM((B_PER_W,), jnp.int32),
        pltpu.VMEM((CHUNK, DIM), jnp.float32),
        pltpu.VMEM((CHUNK, DIM), jnp.float32),
        pltpu.SemaphoreType.DMA,
        pltpu.SemaphoreType.DMA,
        pltpu.SemaphoreType.DMA,
        pltpu.SemaphoreType.DMA,
    ],
    compiler_params=pltpu.CompilerParams(use_tc_tiling_on_sc=False),
)
def _gather_kernel(idx_hbm, table_hbm, out_hbm, idx_all, rows0, rows1,
                   sg0, sg1, ss0, ss1):
    wid = lax.axis_index("s") * NC + lax.axis_index("c")
    wbase = wid * B_PER_W
    rows = [rows0, rows1]
    sg = [sg0, sg1]
    ss = [ss0, ss1]

    pltpu.sync_copy(idx_hbm.at[pl.ds(wbase, B_PER_W)], idx_all)

    def idx_of(c):
        return idx_all.at[pl.ds(c * CHUNK, CHUNK)]

    def out_of(c):
        return out_hbm.at[pl.ds(wbase + c * CHUNK, CHUNK)]

    pltpu.async_copy(table_hbm.at[idx_of(0)], rows[0], sg[0])

    for c in range(NCHUNK):
        b = c % 2
        o = b ^ 1
        if c + 1 < NCHUNK:
            if c - 1 >= 0:
                pltpu.make_async_copy(rows[o], out_of(c - 1), ss[o]).wait()
            pltpu.async_copy(table_hbm.at[idx_of(c + 1)], rows[o], sg[o])
        pltpu.make_async_copy(table_hbm.at[idx_of(c)], rows[b], sg[b]).wait()
        pltpu.async_copy(rows[b], out_of(c), ss[b])

    pltpu.make_async_copy(rows[(NCHUNK - 2) % 2], out_of(NCHUNK - 2),
                          ss[(NCHUNK - 2) % 2]).wait()
    pltpu.make_async_copy(rows[(NCHUNK - 1) % 2], out_of(NCHUNK - 1),
                          ss[(NCHUNK - 1) % 2]).wait()


def kernel(X, weight):
    idx = jnp.transpose(X).reshape(-1).astype(jnp.int32)
    out = _gather_kernel(idx, weight)
    return jnp.transpose(out.reshape(COLS, ROWS, DIM), (1, 0, 2))
